# Initial kernel scaffold; baseline (speedup 1.0000x reference)
import jax
import jax.numpy as jnp
from jax.experimental import pallas as pl


def kernel(graph, nfeat, edge_weight, W_enc, b_enc, W1, b1, W2, b2):
    raise NotImplementedError("write your pallas kernel here")


# speedup vs baseline: 6.8288x; 6.8288x over previous
import functools

import jax
import jax.numpy as jnp
from jax import lax
from jax.experimental import pallas as pl
from jax.experimental.pallas import tpu as pltpu
from jax.experimental.pallas import tpu_sc as plsc

N = 10000
NP = 10240
D = 128
E = 320000
NC = 2
NS = 16
L = 16
NW = NC * NS
EPT = E // NW
C = 80
NCH = EPT // C
RPT = NP // NS
DW = 128

_MESH = plsc.VectorSubcoreMesh(core_axis_name="c", subcore_axis_name="s")

_GDN = lax.GatherDimensionNumbers(
    offset_dims=(), collapsed_slice_dims=(0,), start_index_map=(0,))


def _splat(vec, i):
    idx = jnp.full((L, 1), i, jnp.int32)
    return lax.gather(vec, idx, _GDN, slice_sizes=(1,),
                      mode=lax.GatherScatterMode.PROMISE_IN_BOUNDS)


@functools.partial(
    pl.kernel,
    out_type=jax.ShapeDtypeStruct((NC, NP, DW), jnp.float32),
    mesh=_MESH,
    scratch_types=[
        pltpu.VMEM_SHARED((NP, DW), jnp.float32),
        pltpu.VMEM((C, DW), jnp.float32),
        pltpu.VMEM((C,), jnp.int32),
        pltpu.SemaphoreType.DMA,
    ],
)
def _deg_kernel(dst_hbm, ones_hbm, zeros_hbm, out_hbm, acc, ones, didx, sem):
    cid = lax.axis_index("c")
    sid = lax.axis_index("s")
    wid = sid * NC + cid

    pltpu.sync_copy(ones_hbm, ones)
    base = sid * RPT
    pltpu.sync_copy(zeros_hbm.at[pl.ds(base, RPT)], acc.at[pl.ds(base, RPT)])
    plsc.subcore_barrier()

    def chunk(ci, carry):
        off = pl.multiple_of(wid * EPT + ci * C, 8)
        pltpu.sync_copy(dst_hbm.at[pl.ds(off, C)], didx)
        pltpu.sync_copy(ones, acc.at[didx], add=True)
        return carry

    lax.fori_loop(0, NCH, chunk, 0)
    plsc.subcore_barrier()
    pltpu.sync_copy(acc.at[pl.ds(base, RPT)], out_hbm.at[cid, pl.ds(base, RPT)])


@functools.partial(
    pl.kernel,
    out_type=jax.ShapeDtypeStruct((NC, NP, D), jnp.float32),
    mesh=_MESH,
    scratch_types=[
        pltpu.VMEM_SHARED((NP, D), jnp.float32),
        pltpu.VMEM((C, D), jnp.float32),
        pltpu.VMEM((C,), jnp.int32),
        pltpu.VMEM((C,), jnp.int32),
        pltpu.VMEM((C,), jnp.float32),
        pltpu.SemaphoreType.DMA,
    ],
)
def _agg_kernel(src_hbm, dst_hbm, ew_hbm, h_hbm, zeros_hbm, out_hbm,
                acc, rows, sidx, didx, ewb, sem):
    cid = lax.axis_index("c")
    sid = lax.axis_index("s")
    wid = sid * NC + cid

    base = sid * RPT
    pltpu.sync_copy(zeros_hbm.at[pl.ds(base, RPT)], acc.at[pl.ds(base, RPT)])
    plsc.subcore_barrier()

    def chunk(ci, carry):
        off = pl.multiple_of(wid * EPT + ci * C, 8)
        pltpu.sync_copy(src_hbm.at[pl.ds(off, C)], sidx)
        pltpu.sync_copy(dst_hbm.at[pl.ds(off, C)], didx)
        pltpu.sync_copy(ew_hbm.at[pl.ds(off, C)], ewb)
        pltpu.async_copy(h_hbm.at[sidx], rows, sem).wait()
        for g in range(C // L):
            ew16 = ewb[pl.ds(g * L, L)]
            for i in range(L):
                s = _splat(ew16, i)
                e = g * L + i
                for j in range(D // L):
                    rows[e, pl.ds(j * L, L)] = rows[e, pl.ds(j * L, L)] * s
        pltpu.sync_copy(rows, acc.at[didx], add=True)
        return carry

    lax.fori_loop(0, NCH, chunk, 0)
    plsc.subcore_barrier()
    pltpu.sync_copy(acc.at[pl.ds(base, RPT)], out_hbm.at[cid, pl.ds(base, RPT)])


def _ln(t, eps=1e-5):
    mu = jnp.mean(t, axis=-1, keepdims=True)
    var = jnp.mean((t - mu) ** 2, axis=-1, keepdims=True)
    return (t - mu) * lax.rsqrt(var + eps)


def _tc1_body(degp_ref, x_ref, w_ref, dinv_ref, h_ref):
    deg = degp_ref[0, :N, 0:1] + degp_ref[1, :N, 0:1]
    dinv = lax.rsqrt(jnp.maximum(deg, 1.0))
    dinv_ref[...] = dinv
    h = jnp.dot(x_ref[...], w_ref[...], preferred_element_type=jnp.float32,
                precision=lax.Precision.HIGHEST)
    h_ref[...] = h * dinv


_tc1 = pl.pallas_call(
    _tc1_body,
    out_shape=(jax.ShapeDtypeStruct((N, 1), jnp.float32),
               jax.ShapeDtypeStruct((N, D), jnp.float32)),
)


def _tc2_body(p_ref, dinv_ref, b_ref, w_ref, h_ref):
    agg = p_ref[0, :N] + p_ref[1, :N]
    dinv = dinv_ref[...]
    x = jnp.maximum(agg * dinv + b_ref[...], 0.0)
    h_ref[...] = jnp.dot(x, w_ref[...], preferred_element_type=jnp.float32,
                         precision=lax.Precision.HIGHEST) * dinv


_tc2 = pl.pallas_call(
    _tc2_body,
    out_shape=jax.ShapeDtypeStruct((N, D), jnp.float32),
)


def _tc3_body(p_ref, dinv_ref, b_ref, w_ref, h_ref):
    agg = p_ref[0, :N] + p_ref[1, :N]
    dinv = dinv_ref[...]
    x = jnp.maximum(_ln(agg * dinv + b_ref[...]), 0.0)
    h_ref[...] = jnp.dot(x, w_ref[...], preferred_element_type=jnp.float32,
                         precision=lax.Precision.HIGHEST) * dinv


_tc3 = pl.pallas_call(
    _tc3_body,
    out_shape=jax.ShapeDtypeStruct((N, D), jnp.float32),
)


def _tc4_body(p_ref, dinv_ref, b_ref, out_ref):
    agg = p_ref[0, :N] + p_ref[1, :N]
    out_ref[...] = _ln(agg * dinv_ref[...] + b_ref[...])


_tc4 = pl.pallas_call(
    _tc4_body,
    out_shape=jax.ShapeDtypeStruct((N, D), jnp.float32),
)


def kernel(graph, nfeat, edge_weight, W_enc, b_enc, W1, b1, W2, b2):
    src = graph[0]
    dst = graph[1]
    ones_rows = jnp.ones((C, DW), jnp.float32)
    zeros_deg = jnp.zeros((NP, DW), jnp.float32)
    zeros_agg = jnp.zeros((NP, D), jnp.float32)

    degp = _deg_kernel(dst, ones_rows, zeros_deg)
    dinv, h0 = _tc1(degp, nfeat, W_enc)

    p = _agg_kernel(src, dst, edge_weight, h0, zeros_agg)
    h1 = _tc2(p, dinv, b_enc.reshape(1, D), W1)

    p = _agg_kernel(src, dst, edge_weight, h1, zeros_agg)
    h2 = _tc3(p, dinv, b1.reshape(1, D), W2)

    p = _agg_kernel(src, dst, edge_weight, h2, zeros_agg)
    return _tc4(p, dinv, b2.reshape(1, D))


# speedup vs baseline: 12.4835x; 1.8281x over previous
import functools

import jax
import jax.numpy as jnp
from jax import lax
from jax.experimental import pallas as pl
from jax.experimental.pallas import tpu as pltpu
from jax.experimental.pallas import tpu_sc as plsc

N = 10000
NP = 10240
D = 128
E = 320000
NC = 2
NS = 16
L = 16
NW = NC * NS
EPT = E // NW
C = 80
NCH = EPT // C
RPT = NP // NS
DW = 128

_MESH = plsc.VectorSubcoreMesh(core_axis_name="c", subcore_axis_name="s")

_GDN = lax.GatherDimensionNumbers(
    offset_dims=(), collapsed_slice_dims=(0,), start_index_map=(0,))


def _splat(vec, i):
    idx = jnp.full((L, 1), i, jnp.int32)
    return lax.gather(vec, idx, _GDN, slice_sizes=(1,),
                      mode=lax.GatherScatterMode.PROMISE_IN_BOUNDS)


@functools.partial(
    pl.kernel,
    out_type=jax.ShapeDtypeStruct((NC, NP, DW), jnp.float32),
    mesh=_MESH,
    scratch_types=[
        pltpu.VMEM_SHARED((NP, DW), jnp.float32),
        pltpu.VMEM((C, DW), jnp.float32),
        pltpu.VMEM((C,), jnp.int32),
        pltpu.SemaphoreType.DMA,
    ],
)
def _deg_kernel(dst_hbm, ones_hbm, zeros_hbm, out_hbm, acc, ones, didx, sem):
    cid = lax.axis_index("c")
    sid = lax.axis_index("s")
    wid = sid * NC + cid

    pltpu.sync_copy(ones_hbm, ones)
    base = sid * RPT
    pltpu.sync_copy(zeros_hbm.at[pl.ds(base, RPT)], acc.at[pl.ds(base, RPT)])
    plsc.subcore_barrier()

    def chunk(ci, carry):
        off = pl.multiple_of(wid * EPT + ci * C, 8)
        pltpu.sync_copy(dst_hbm.at[pl.ds(off, C)], didx)
        pltpu.sync_copy(ones, acc.at[didx], add=True)
        return carry

    lax.fori_loop(0, NCH, chunk, 0)
    plsc.subcore_barrier()
    pltpu.sync_copy(acc.at[pl.ds(base, RPT)], out_hbm.at[cid, pl.ds(base, RPT)])


@functools.partial(
    pl.kernel,
    out_type=jax.ShapeDtypeStruct((NC, NP, D), jnp.float32),
    mesh=_MESH,
    scratch_types=[
        pltpu.VMEM_SHARED((NP, D), jnp.float32),
        pltpu.VMEM((2, C, D), jnp.float32),
        pltpu.VMEM((2, 1, C), jnp.int32),
        pltpu.VMEM((2, 1, C), jnp.int32),
        pltpu.VMEM((2, 1, C), jnp.float32),
        pltpu.SemaphoreType.DMA,
        pltpu.SemaphoreType.DMA,
        pltpu.SemaphoreType.DMA,
        pltpu.SemaphoreType.DMA,
    ],
)
def _agg_kernel(src_hbm, dst_hbm, ew_hbm, h_hbm, zeros_hbm, out_hbm,
                acc, rows, sidxb, didxb, ewb, semG0, semG1, semX0, semX1):
    cid = lax.axis_index("c")
    sid = lax.axis_index("s")
    wid = sid * NC + cid

    base = sid * RPT
    pltpu.sync_copy(zeros_hbm.at[pl.ds(base, RPT)], acc.at[pl.ds(base, RPT)])
    plsc.subcore_barrier()

    def scale(b):
        for g in range(C // L):
            ew16 = ewb[b, 0, pl.ds(g * L, L)]
            for i in range(L):
                s = _splat(ew16, i)
                e = g * L + i
                for j in range(D // L):
                    rows[b, e, pl.ds(j * L, L)] = rows[b, e, pl.ds(j * L, L)] * s

    def load_idx(ci, b, sem):
        pltpu.async_copy(src_hbm.at[wid, ci], sidxb.at[b], sem)
        pltpu.async_copy(dst_hbm.at[wid, ci], didxb.at[b], sem)
        pltpu.async_copy(ew_hbm.at[wid, ci], ewb.at[b], sem)

    def wait_idx(ci, b, sem):
        pltpu.make_async_copy(src_hbm.at[wid, ci], sidxb.at[b], sem).wait()
        pltpu.make_async_copy(dst_hbm.at[wid, ci], didxb.at[b], sem).wait()
        pltpu.make_async_copy(ew_hbm.at[wid, ci], ewb.at[b], sem).wait()

    load_idx(0, 0, semX0)
    wait_idx(0, 0, semX0)
    pltpu.async_copy(h_hbm.at[sidxb.at[0, 0]], rows.at[0], semG0)
    load_idx(1, 1, semX1)

    def pair(p, carry):
        ci0 = p * 2
        ci1 = ci0 + 1
        pltpu.make_async_copy(h_hbm.at[sidxb.at[0, 0]], rows.at[0], semG0).wait()
        wait_idx(ci1, 1, semX1)
        pltpu.async_copy(h_hbm.at[sidxb.at[1, 0]], rows.at[1], semG1)
        scale(0)
        pltpu.sync_copy(rows.at[0], acc.at[didxb.at[0, 0]], add=True)
        load_idx(ci0 + 2, 0, semX0)
        pltpu.make_async_copy(h_hbm.at[sidxb.at[1, 0]], rows.at[1], semG1).wait()
        wait_idx(ci0 + 2, 0, semX0)
        pltpu.async_copy(h_hbm.at[sidxb.at[0, 0]], rows.at[0], semG0)
        scale(1)
        pltpu.sync_copy(rows.at[1], acc.at[didxb.at[1, 0]], add=True)

        @pl.when(ci0 + 3 < NCH)
        def _():
            load_idx(ci0 + 3, 1, semX1)
        return carry

    lax.fori_loop(0, (NCH - 1) // 2, pair, 0)

    pltpu.make_async_copy(h_hbm.at[sidxb.at[0, 0]], rows.at[0], semG0).wait()
    scale(0)
    pltpu.sync_copy(rows.at[0], acc.at[didxb.at[0, 0]], add=True)

    plsc.subcore_barrier()
    pltpu.sync_copy(acc.at[pl.ds(base, RPT)], out_hbm.at[cid, pl.ds(base, RPT)])


def _ln(t, eps=1e-5):
    mu = jnp.mean(t, axis=-1, keepdims=True)
    var = jnp.mean((t - mu) ** 2, axis=-1, keepdims=True)
    return (t - mu) * lax.rsqrt(var + eps)


def _tc1_body(degp_ref, x_ref, w_ref, dinv_ref, h_ref):
    deg = degp_ref[0, :N, 0:1] + degp_ref[1, :N, 0:1]
    dinv = lax.rsqrt(jnp.maximum(deg, 1.0))
    dinv_ref[...] = dinv
    h = jnp.dot(x_ref[...], w_ref[...], preferred_element_type=jnp.float32,
                precision=lax.Precision.HIGHEST)
    h_ref[...] = h * dinv


_tc1 = pl.pallas_call(
    _tc1_body,
    out_shape=(jax.ShapeDtypeStruct((N, 1), jnp.float32),
               jax.ShapeDtypeStruct((N, D), jnp.float32)),
)


def _tc2_body(p_ref, dinv_ref, b_ref, w_ref, h_ref):
    agg = p_ref[0, :N] + p_ref[1, :N]
    dinv = dinv_ref[...]
    x = jnp.maximum(agg * dinv + b_ref[...], 0.0)
    h_ref[...] = jnp.dot(x, w_ref[...], preferred_element_type=jnp.float32,
                         precision=lax.Precision.HIGHEST) * dinv


_tc2 = pl.pallas_call(
    _tc2_body,
    out_shape=jax.ShapeDtypeStruct((N, D), jnp.float32),
)


def _tc3_body(p_ref, dinv_ref, b_ref, w_ref, h_ref):
    agg = p_ref[0, :N] + p_ref[1, :N]
    dinv = dinv_ref[...]
    x = jnp.maximum(_ln(agg * dinv + b_ref[...]), 0.0)
    h_ref[...] = jnp.dot(x, w_ref[...], preferred_element_type=jnp.float32,
                         precision=lax.Precision.HIGHEST) * dinv


_tc3 = pl.pallas_call(
    _tc3_body,
    out_shape=jax.ShapeDtypeStruct((N, D), jnp.float32),
)


def _tc4_body(p_ref, dinv_ref, b_ref, out_ref):
    agg = p_ref[0, :N] + p_ref[1, :N]
    out_ref[...] = _ln(agg * dinv_ref[...] + b_ref[...])


_tc4 = pl.pallas_call(
    _tc4_body,
    out_shape=jax.ShapeDtypeStruct((N, D), jnp.float32),
)


def kernel(graph, nfeat, edge_weight, W_enc, b_enc, W1, b1, W2, b2):
    src = graph[0]
    dst = graph[1]
    ones_rows = jnp.ones((C, DW), jnp.float32)
    zeros_deg = jnp.zeros((NP, DW), jnp.float32)
    zeros_agg = jnp.zeros((NP, D), jnp.float32)

    degp = _deg_kernel(dst, ones_rows, zeros_deg)
    dinv, h0 = _tc1(degp, nfeat, W_enc)

    src4 = src.reshape(NW, NCH, 1, C)
    dst4 = dst.reshape(NW, NCH, 1, C)
    ew4 = edge_weight.reshape(NW, NCH, 1, C)

    p = _agg_kernel(src4, dst4, ew4, h0, zeros_agg)
    h1 = _tc2(p, dinv, b_enc.reshape(1, D), W1)

    p = _agg_kernel(src4, dst4, ew4, h1, zeros_agg)
    h2 = _tc3(p, dinv, b1.reshape(1, D), W2)

    p = _agg_kernel(src4, dst4, ew4, h2, zeros_agg)
    return _tc4(p, dinv, b2.reshape(1, D))
